# baseline (speedup 1.0000x reference)
import jax
import jax.numpy as jnp
from jax import lax
from jax.experimental import pallas as pl
from jax.experimental.pallas import tpu as pltpu
from jax.experimental.pallas import tpu_sc as plsc

import functools


@functools.partial(jax.jit, static_argnames=("batch", "seq_len", "hidden"))
def _gather_last(seq3, lengths, *, batch, seq_len, hidden):
    half = hidden // 2
    mesh = plsc.ScalarSubcoreMesh(axis_name="c", num_cores=2)

    @functools.partial(
        pl.kernel,
        out_type=jax.ShapeDtypeStruct((batch, hidden), jnp.float32),
        mesh=mesh,
        scratch_types=[
            pltpu.SMEM((batch,), jnp.int32),
            pltpu.SemaphoreType.DMA,
        ],
    )
    def k(seq3_hbm, len_hbm, out_hbm, len_sm, sem):
        core = lax.axis_index("c")
        hb = batch // 2

        @pl.when(core == 0)
        def _c0():
            bw = pltpu.async_copy(
                seq3_hbm.at[:, 0, pl.ds(half, half)],
                out_hbm.at[:, pl.ds(half, half)],
                sem,
            )
            pltpu.sync_copy(len_hbm, len_sm)
            copies = []
            for b in range(hb):
                t = len_sm[b] - 1
                copies.append(
                    pltpu.async_copy(
                        seq3_hbm.at[b, pl.ds(t, 1), pl.ds(0, half)],
                        out_hbm.at[pl.ds(b, 1), pl.ds(0, half)],
                        sem,
                    )
                )
            bw.wait()
            for c in copies:
                c.wait()

        @pl.when(core == 1)
        def _c1():
            pltpu.sync_copy(len_hbm, len_sm)
            copies = []
            for b in range(hb, batch):
                t = len_sm[b] - 1
                copies.append(
                    pltpu.async_copy(
                        seq3_hbm.at[b, pl.ds(t, 1), pl.ds(0, half)],
                        out_hbm.at[pl.ds(b, 1), pl.ds(0, half)],
                        sem,
                    )
                )
            for c in copies:
                c.wait()

    return k(seq3, lengths)


def kernel(sequences, lengths):
    batch, seq_len, hidden_x_dirs = sequences.shape
    return _gather_last(
        sequences,
        lengths.astype(jnp.int32),
        batch=batch,
        seq_len=seq_len,
        hidden=hidden_x_dirs,
    )


# speedup vs baseline: 1.0678x; 1.0678x over previous
import jax
import jax.numpy as jnp
from jax import lax
from jax.experimental import pallas as pl
from jax.experimental.pallas import tpu as pltpu
from jax.experimental.pallas import tpu_sc as plsc

import functools


@functools.partial(jax.jit, static_argnames=("batch", "seq_len", "hidden"))
def _gather_last(seq3, lengths, *, batch, seq_len, hidden):
    half = hidden // 2
    mesh = plsc.ScalarSubcoreMesh(axis_name="c", num_cores=1)

    @functools.partial(
        pl.kernel,
        out_type=jax.ShapeDtypeStruct((batch, hidden), jnp.float32),
        mesh=mesh,
        scratch_types=[
            pltpu.SMEM((batch,), jnp.int32),
            pltpu.SemaphoreType.DMA,
        ],
    )
    def k(seq3_hbm, len_hbm, out_hbm, len_sm, sem):
        bw = pltpu.async_copy(
            seq3_hbm.at[:, 0, pl.ds(half, half)],
            out_hbm.at[:, pl.ds(half, half)],
            sem,
        )
        pltpu.sync_copy(len_hbm, len_sm)
        copies = []
        for b in range(batch):
            t = len_sm[b] - 1
            copies.append(
                pltpu.async_copy(
                    seq3_hbm.at[b, pl.ds(t, 1), pl.ds(0, half)],
                    out_hbm.at[pl.ds(b, 1), pl.ds(0, half)],
                    sem,
                )
            )
        bw.wait()
        for c in copies:
            c.wait()

    return k(seq3, lengths)


def kernel(sequences, lengths):
    batch, seq_len, hidden_x_dirs = sequences.shape
    return _gather_last(
        sequences,
        lengths.astype(jnp.int32),
        batch=batch,
        seq_len=seq_len,
        hidden=hidden_x_dirs,
    )
